# baseline (speedup 1.0000x reference)
import functools

import jax
import jax.numpy as jnp
from jax import lax
from jax.experimental import pallas as pl
from jax.experimental.pallas import tpu as pltpu
from jax.experimental.pallas import tpu_sc as plsc

_B = 16384
_D = 16
_EPS = 1e-05
_NC = 2
_NS = 16
_NW = _NC * _NS
_BPW = _B // _NW
_CH = 256
_V = 1000000
_VB = _V // 8
_CBLK = 1000


def _compact_body(i_ref, o_ref):
    o_ref[...] = jnp.concatenate([i_ref[:, s, :] for s in range(8)], axis=1)


def _compact(theta):
    th3 = theta.reshape(_VB, 8, _D)
    return pl.pallas_call(
        _compact_body,
        grid=(_VB // _CBLK,),
        in_specs=[pl.BlockSpec((_CBLK, 8, _D), lambda i: (i, 0, 0))],
        out_specs=pl.BlockSpec((_CBLK, 128), lambda i: (i, 0)),
        out_shape=jax.ShapeDtypeStruct((_VB, 128), jnp.float32),
    )(th3)


def _make_sc_main():
    mesh = plsc.VectorSubcoreMesh(core_axis_name="c", subcore_axis_name="s")

    @functools.partial(
        pl.kernel,
        out_type=[
            jax.ShapeDtypeStruct((_B,), jnp.float32),
            jax.ShapeDtypeStruct((_B,), jnp.float32),
            jax.ShapeDtypeStruct((_B,), jnp.float32),
        ],
        mesh=mesh,
        scratch_types=[
            pltpu.VMEM((_BPW,), jnp.int32),
            pltpu.VMEM((_BPW,), jnp.int32),
            pltpu.VMEM((_CH,), jnp.int32),
            pltpu.VMEM((_CH, 128), jnp.float32),
            pltpu.VMEM((_CH, 128), jnp.float32),
            pltpu.VMEM((_BPW,), jnp.float32),
            pltpu.VMEM((_BPW,), jnp.float32),
            pltpu.VMEM((_BPW,), jnp.float32),
            pltpu.SemaphoreType.DMA,
            pltpu.SemaphoreType.DMA,
        ],
        compiler_params=pltpu.CompilerParams(needs_layout_passes=False),
    )
    def sc_main(u_hbm, v_hbm, th_hbm, su_hbm, sv_hbm, sd_hbm,
                iu_v, iv_v, blk_v, ue_v, ve_v, su_v, sv_v, sd_v,
                sem, sem2):
        wid = lax.axis_index("s") * _NC + lax.axis_index("c")
        base = wid * _BPW
        pltpu.sync_copy(u_hbm.at[pl.ds(base, _BPW)], iu_v)
        pltpu.sync_copy(v_hbm.at[pl.ds(base, _BPW)], iv_v)

        for h in range(_BPW // _CH):
            off = h * _CH

            @pl.loop(0, _CH // 16)
            def _mku(b):
                blk_v[pl.ds(b * 16, 16)] = lax.shift_right_logical(
                    iu_v[pl.ds(off + b * 16, 16)], 3)

            pltpu.async_copy(th_hbm.at[blk_v], ue_v, sem).wait()

            @pl.loop(0, _CH // 16)
            def _mkv(b):
                blk_v[pl.ds(b * 16, 16)] = lax.shift_right_logical(
                    iv_v[pl.ds(off + b * 16, 16)], 3)

            pltpu.async_copy(th_hbm.at[blk_v], ve_v, sem2).wait()

            @pl.loop(0, _CH // 16)
            def _reduce(b):
                rows = lax.iota(jnp.int32, 16) + b * 16
                cu0 = (iu_v[pl.ds(off + b * 16, 16)] & 7) * 16
                cv0 = (iv_v[pl.ds(off + b * 16, 16)] & 7) * 16
                su = jnp.zeros((16,), jnp.float32)
                sv = jnp.zeros((16,), jnp.float32)
                sd = jnp.zeros((16,), jnp.float32)
                for d in range(_D):
                    cu = plsc.load_gather(ue_v, [rows, cu0 + d])
                    cv = plsc.load_gather(ve_v, [rows, cv0 + d])
                    su = su + cu * cu
                    sv = sv + cv * cv
                    dd = cu - cv
                    sd = sd + dd * dd
                su_v[pl.ds(off + b * 16, 16)] = su
                sv_v[pl.ds(off + b * 16, 16)] = sv
                sd_v[pl.ds(off + b * 16, 16)] = sd

        pltpu.sync_copy(su_v, su_hbm.at[pl.ds(base, _BPW)])
        pltpu.sync_copy(sv_v, sv_hbm.at[pl.ds(base, _BPW)])
        pltpu.sync_copy(sd_v, sd_hbm.at[pl.ds(base, _BPW)])

    return sc_main


def _tc_tail_body(r_ref, t_ref, su_ref, sv_ref, sd_ref, o_ref):
    su = jnp.clip(su_ref[...], 0.0, 1.0 - _EPS)
    sv = jnp.clip(sv_ref[...], 0.0, 1.0 - _EPS)
    nrm = jnp.sqrt(sd_ref[...] + _EPS)
    zm1 = 2.0 * nrm / ((1.0 - su) * (1.0 - sv))
    duv = jnp.log((1.0 + zm1) + jnp.sqrt(zm1 * (zm1 + 2.0)))
    r = r_ref[0, 0]
    t = t_ref[0, 0]
    o_ref[...] = 1.0 / (jnp.exp((duv - r) / t) + 1.0)


def _tc_tail(su, sv, sd, r, t):
    return pl.pallas_call(
        _tc_tail_body,
        in_specs=[
            pl.BlockSpec(memory_space=pltpu.SMEM),
            pl.BlockSpec(memory_space=pltpu.SMEM),
            pl.BlockSpec((128, 128), lambda: (0, 0)),
            pl.BlockSpec((128, 128), lambda: (0, 0)),
            pl.BlockSpec((128, 128), lambda: (0, 0)),
        ],
        out_specs=pl.BlockSpec((128, 128), lambda: (0, 0)),
        out_shape=jax.ShapeDtypeStruct((128, 128), jnp.float32),
    )(r.reshape(1, 1), t.reshape(1, 1),
      su.reshape(128, 128), sv.reshape(128, 128), sd.reshape(128, 128))


def kernel(u, v, theta, r, t):
    u = u.astype(jnp.int32)
    v = v.astype(jnp.int32)
    r = jnp.asarray(r, jnp.float32)
    t = jnp.asarray(t, jnp.float32)
    th128 = _compact(theta)
    su, sv, sd = _make_sc_main()(u, v, th128)
    out = _tc_tail(su, sv, sd, r, t)
    return out.reshape(_B)


# speedup vs baseline: 1.0200x; 1.0200x over previous
import functools

import jax
import jax.numpy as jnp
from jax import lax
from jax.experimental import pallas as pl
from jax.experimental.pallas import tpu as pltpu
from jax.experimental.pallas import tpu_sc as plsc

_B = 16384
_D = 16
_EPS = 1e-05
_NC = 2
_NS = 16
_NW = _NC * _NS
_BPW = _B // _NW


def _make_sc_main():
    mesh = plsc.VectorSubcoreMesh(core_axis_name="c", subcore_axis_name="s")

    @functools.partial(
        pl.kernel,
        out_type=[
            jax.ShapeDtypeStruct((_B,), jnp.float32),
            jax.ShapeDtypeStruct((_B,), jnp.float32),
            jax.ShapeDtypeStruct((_B,), jnp.float32),
        ],
        mesh=mesh,
        scratch_types=[
            pltpu.VMEM((_BPW,), jnp.int32),
            pltpu.VMEM((_BPW, _D), jnp.float32),
            pltpu.VMEM((_BPW, _D), jnp.float32),
            pltpu.VMEM((_BPW,), jnp.float32),
            pltpu.VMEM((_BPW,), jnp.float32),
            pltpu.VMEM((_BPW,), jnp.float32),
            pltpu.SemaphoreType.DMA,
        ],
        compiler_params=pltpu.CompilerParams(
            use_tc_tiling_on_sc=False, needs_layout_passes=False),
    )
    def sc_main(u_hbm, v_hbm, th_hbm, su_hbm, sv_hbm, sd_hbm,
                idx_v, ue_v, ve_v, su_v, sv_v, sd_v, sem):
        wid = lax.axis_index("s") * _NC + lax.axis_index("c")
        base = wid * _BPW
        pltpu.sync_copy(u_hbm.at[pl.ds(base, _BPW)], idx_v)
        pltpu.async_copy(th_hbm.at[idx_v], ue_v, sem).wait()
        pltpu.sync_copy(v_hbm.at[pl.ds(base, _BPW)], idx_v)
        pltpu.async_copy(th_hbm.at[idx_v], ve_v, sem).wait()

        @pl.loop(0, _BPW // 16)
        def _reduce(b):
            rows = lax.iota(jnp.int32, 16) + b * 16
            su = jnp.zeros((16,), jnp.float32)
            sv = jnp.zeros((16,), jnp.float32)
            sd = jnp.zeros((16,), jnp.float32)
            for d in range(_D):
                cols = jnp.full((16,), d, jnp.int32)
                cu = plsc.load_gather(ue_v, [rows, cols])
                cv = plsc.load_gather(ve_v, [rows, cols])
                su = su + cu * cu
                sv = sv + cv * cv
                dd = cu - cv
                sd = sd + dd * dd
            su_v[pl.ds(b * 16, 16)] = su
            sv_v[pl.ds(b * 16, 16)] = sv
            sd_v[pl.ds(b * 16, 16)] = sd

        pltpu.sync_copy(su_v, su_hbm.at[pl.ds(base, _BPW)])
        pltpu.sync_copy(sv_v, sv_hbm.at[pl.ds(base, _BPW)])
        pltpu.sync_copy(sd_v, sd_hbm.at[pl.ds(base, _BPW)])

    return sc_main


def _tc_tail_body(r_ref, t_ref, su_ref, sv_ref, sd_ref, o_ref):
    su = jnp.clip(su_ref[...], 0.0, 1.0 - _EPS)
    sv = jnp.clip(sv_ref[...], 0.0, 1.0 - _EPS)
    nrm = jnp.sqrt(sd_ref[...] + _EPS)
    zm1 = 2.0 * nrm / ((1.0 - su) * (1.0 - sv))
    duv = jnp.log((1.0 + zm1) + jnp.sqrt(zm1 * (zm1 + 2.0)))
    r = r_ref[0, 0]
    t = t_ref[0, 0]
    o_ref[...] = 1.0 / (jnp.exp((duv - r) / t) + 1.0)


def _tc_tail(su, sv, sd, r, t):
    return pl.pallas_call(
        _tc_tail_body,
        in_specs=[
            pl.BlockSpec(memory_space=pltpu.SMEM),
            pl.BlockSpec(memory_space=pltpu.SMEM),
            pl.BlockSpec((128, 128), lambda: (0, 0)),
            pl.BlockSpec((128, 128), lambda: (0, 0)),
            pl.BlockSpec((128, 128), lambda: (0, 0)),
        ],
        out_specs=pl.BlockSpec((128, 128), lambda: (0, 0)),
        out_shape=jax.ShapeDtypeStruct((128, 128), jnp.float32),
    )(r.reshape(1, 1), t.reshape(1, 1),
      su.reshape(128, 128), sv.reshape(128, 128), sd.reshape(128, 128))


def kernel(u, v, theta, r, t):
    u = u.astype(jnp.int32)
    v = v.astype(jnp.int32)
    r = jnp.asarray(r, jnp.float32)
    t = jnp.asarray(t, jnp.float32)
    su, sv, sd = _make_sc_main()(u, v, theta)
    out = _tc_tail(su, sv, sd, r, t)
    return out.reshape(_B)
